# baseline (speedup 1.0000x reference)
import functools

import jax
import jax.numpy as jnp
from jax import lax
from jax.experimental import pallas as pl
from jax.experimental.pallas import tpu as pltpu
from jax.experimental.pallas import tpu_sc as plsc

N = 320000
D = 128
NUM_SEG = 10000
SEG_PAD = 10240
GROUP = 128
NGROUPS = N // GROUP
LANES = 16

_info = plsc.get_sparse_core_info()
NC = _info.num_cores
NS = _info.num_subcores
NW = NC * NS
SEG_PER_TILE = SEG_PAD // NS
TRIPS = -(-NGROUPS // NW)
TRIPS += TRIPS % 2


def _phase1_body(x_hbm, seg_hbm, zsum_hbm, sums_hbm, cnts_hbm,
                 dbuf0, dbuf1, idx0, idx1, ones_v, zcnt, acc_sp, cnt_sp,
                 sem_d0, sem_i0, sem_d1, sem_i1):
    c = lax.axis_index("c")
    s = lax.axis_index("s")
    wid = s * NC + c

    zero16 = jnp.zeros((LANES,), jnp.float32)
    one16 = jnp.ones((LANES,), jnp.float32)

    def init_zcnt(i, _):
        zcnt[pl.ds(i * LANES, LANES)] = zero16
        return 0
    lax.fori_loop(0, SEG_PER_TILE // LANES, init_zcnt, 0)

    def init_ones(i, _):
        ones_v[pl.ds(i * LANES, LANES)] = one16
        return 0
    lax.fori_loop(0, GROUP // LANES, init_ones, 0)

    gs = (wid * NGROUPS) // NW
    ge = ((wid + 1) * NGROUPS) // NW
    ng = ge - gs

    slots = ((dbuf0, idx0, sem_d0, sem_i0), (dbuf1, idx1, sem_d1, sem_i1))

    def start(i, slot):
        dbuf_b, idx_b, sem_d, sem_i = slot
        gi = jnp.where(i < ng, gs + i, gs)
        off = gi * GROUP
        pltpu.async_copy(x_hbm.at[pl.ds(off, GROUP)], dbuf_b, sem_d)
        pltpu.async_copy(seg_hbm.at[pl.ds(off, GROUP)], idx_b, sem_i)

    def finish(i, slot):
        dbuf_b, idx_b, sem_d, sem_i = slot
        pltpu.make_async_copy(seg_hbm.at[pl.ds(0, GROUP)], idx_b, sem_i).wait()

        @pl.when(i >= ng)
        def _():
            pad = jnp.full((LANES,), NUM_SEG, jnp.int32)
            for j in range(GROUP // LANES):
                idx_b[pl.ds(j * LANES, LANES)] = pad

        pltpu.make_async_copy(x_hbm.at[pl.ds(0, GROUP)], dbuf_b, sem_d).wait()
        pltpu.sync_copy(dbuf_b, acc_sp.at[idx_b], add=True)
        pltpu.sync_copy(ones_v, cnt_sp.at[idx_b], add=True)

    start(0, slots[0])
    start(1, slots[1])

    lo = s * SEG_PER_TILE
    pltpu.sync_copy(zsum_hbm.at[pl.ds(lo, SEG_PER_TILE)],
                    acc_sp.at[pl.ds(lo, SEG_PER_TILE)])
    pltpu.sync_copy(zcnt, cnt_sp.at[pl.ds(lo, SEG_PER_TILE)])
    plsc.subcore_barrier()

    def pair(it, _):
        base = 2 * it
        finish(base, slots[0])

        @pl.when(base + 2 < TRIPS)
        def _():
            start(base + 2, slots[0])

        finish(base + 1, slots[1])

        @pl.when(base + 3 < TRIPS)
        def _():
            start(base + 3, slots[1])
        return 0
    lax.fori_loop(0, TRIPS // 2, pair, 0)

    plsc.subcore_barrier()

    pltpu.sync_copy(acc_sp.at[pl.ds(lo, SEG_PER_TILE)],
                    sums_hbm.at[c].at[pl.ds(lo, SEG_PER_TILE)])
    pltpu.sync_copy(cnt_sp.at[pl.ds(lo, SEG_PER_TILE)],
                    cnts_hbm.at[c].at[pl.ds(lo, SEG_PER_TILE)])


_phase1 = functools.partial(
    pl.kernel,
    mesh=plsc.VectorSubcoreMesh(core_axis_name="c", subcore_axis_name="s"),
    out_type=[
        jax.ShapeDtypeStruct((NC, SEG_PAD, D), jnp.float32),
        jax.ShapeDtypeStruct((NC, SEG_PAD), jnp.float32),
    ],
    scratch_types=[
        pltpu.VMEM((GROUP, D), jnp.float32),
        pltpu.VMEM((GROUP, D), jnp.float32),
        pltpu.VMEM((GROUP,), jnp.int32),
        pltpu.VMEM((GROUP,), jnp.int32),
        pltpu.VMEM((GROUP,), jnp.float32),
        pltpu.VMEM((SEG_PER_TILE,), jnp.float32),
        pltpu.VMEM_SHARED((SEG_PAD, D), jnp.float32),
        pltpu.VMEM_SHARED((SEG_PAD,), jnp.float32),
        pltpu.SemaphoreType.DMA,
        pltpu.SemaphoreType.DMA,
        pltpu.SemaphoreType.DMA,
        pltpu.SemaphoreType.DMA,
    ],
)(_phase1_body)


def _phase2_body(sm, cc, o):
    c = cc[...]
    cnt = jnp.transpose(c[0:1, :] + c[1:2, :], (1, 0))
    o[...] = (sm[0] + sm[1]) / jnp.maximum(cnt, 1.0)


_BS = 512

_phase2 = pl.pallas_call(
    _phase2_body,
    grid=(SEG_PAD // _BS,),
    in_specs=[
        pl.BlockSpec((NC, _BS, D), lambda i: (0, i, 0)),
        pl.BlockSpec((NC, _BS), lambda i: (0, i)),
    ],
    out_specs=pl.BlockSpec((_BS, D), lambda i: (i, 0)),
    out_shape=jax.ShapeDtypeStruct((NUM_SEG, D), jnp.float32),
)


@jax.jit
def kernel(input, segLabels):
    seg = segLabels.astype(jnp.int32)
    zsum = jnp.zeros((SEG_PAD, D), jnp.float32)
    sums, cnts = _phase1(input, seg, zsum)
    return _phase2(sums, cnts)


# speedup vs baseline: 1.0140x; 1.0140x over previous
import functools

import jax
import jax.numpy as jnp
from jax import lax
from jax.experimental import pallas as pl
from jax.experimental.pallas import tpu as pltpu
from jax.experimental.pallas import tpu_sc as plsc

N = 320000
D = 128
NUM_SEG = 10000
SEG_PAD = 10240
GROUP = 128
NGROUPS = N // GROUP
LANES = 16

_info = plsc.get_sparse_core_info()
NC = _info.num_cores
NS = _info.num_subcores
NW = NC * NS
SEG_PER_TILE = SEG_PAD // NS
TRIPS = -(-NGROUPS // NW)
TRIPS += TRIPS % 2


def _phase1_body(x_hbm, seg_hbm, zsum_hbm, sums_hbm, cnts_hbm,
                 dbuf0, dbuf1, idx0, idx1, idx2, idx3, ones_v, zcnt,
                 acc_sp, cnt_sp,
                 sem_d0, sem_d1, sem_i0, sem_i1, sem_i2, sem_i3):
    c = lax.axis_index("c")
    s = lax.axis_index("s")
    wid = s * NC + c

    zero16 = jnp.zeros((LANES,), jnp.float32)
    one16 = jnp.ones((LANES,), jnp.float32)

    def init_zcnt(i, _):
        zcnt[pl.ds(i * LANES, LANES)] = zero16
        return 0
    lax.fori_loop(0, SEG_PER_TILE // LANES, init_zcnt, 0)

    def init_ones(i, _):
        ones_v[pl.ds(i * LANES, LANES)] = one16
        return 0
    lax.fori_loop(0, GROUP // LANES, init_ones, 0)

    gs = (wid * NGROUPS) // NW
    ge = ((wid + 1) * NGROUPS) // NW
    ng = ge - gs

    dbufs = (dbuf0, dbuf1)
    sem_ds = (sem_d0, sem_d1)
    idxs = (idx0, idx1, idx2, idx3)
    sem_is = (sem_i0, sem_i1, sem_i2, sem_i3)

    def start(i, dbuf_b, sem_d, idx_b, sem_i):
        gi = jnp.where(i < ng, gs + i, gs)
        off = gi * GROUP
        pltpu.async_copy(x_hbm.at[pl.ds(off, GROUP)], dbuf_b, sem_d)
        pltpu.async_copy(seg_hbm.at[pl.ds(off, GROUP)], idx_b, sem_i)

    def step(i, j):
        dbuf_b, sem_d = dbufs[j % 2], sem_ds[j % 2]
        idx_b, sem_i = idxs[j % 4], sem_is[j % 4]
        pltpu.make_async_copy(seg_hbm.at[pl.ds(0, GROUP)], idx_b, sem_i).wait()

        @pl.when(i >= ng)
        def _():
            pad = jnp.full((LANES,), NUM_SEG, jnp.int32)
            for jj in range(GROUP // LANES):
                idx_b[pl.ds(jj * LANES, LANES)] = pad

        pltpu.make_async_copy(x_hbm.at[pl.ds(0, GROUP)], dbuf_b, sem_d).wait()
        pltpu.sync_copy(dbuf_b, acc_sp.at[idx_b], add=True)

        @pl.when(i + 2 < TRIPS)
        def _():
            start(i + 2, dbuf_b, sem_d, idxs[(j + 2) % 4], sem_is[(j + 2) % 4])

        pltpu.sync_copy(ones_v, cnt_sp.at[idx_b], add=True)

    start(0, dbuf0, sem_d0, idx0, sem_i0)
    start(1, dbuf1, sem_d1, idx1, sem_i1)

    lo = s * SEG_PER_TILE
    pltpu.sync_copy(zsum_hbm.at[pl.ds(lo, SEG_PER_TILE)],
                    acc_sp.at[pl.ds(lo, SEG_PER_TILE)])
    pltpu.sync_copy(zcnt, cnt_sp.at[pl.ds(lo, SEG_PER_TILE)])
    plsc.subcore_barrier()

    def quad(it, _):
        base = 4 * it
        for j in range(4):
            step(base + j, j)
        return 0
    lax.fori_loop(0, TRIPS // 4, quad, 0)

    plsc.subcore_barrier()

    pltpu.sync_copy(acc_sp.at[pl.ds(lo, SEG_PER_TILE)],
                    sums_hbm.at[c].at[pl.ds(lo, SEG_PER_TILE)])
    pltpu.sync_copy(cnt_sp.at[pl.ds(lo, SEG_PER_TILE)],
                    cnts_hbm.at[c].at[pl.ds(lo, SEG_PER_TILE)])


_phase1 = functools.partial(
    pl.kernel,
    mesh=plsc.VectorSubcoreMesh(core_axis_name="c", subcore_axis_name="s"),
    out_type=[
        jax.ShapeDtypeStruct((NC, SEG_PAD, D), jnp.float32),
        jax.ShapeDtypeStruct((NC, SEG_PAD), jnp.float32),
    ],
    scratch_types=[
        pltpu.VMEM((GROUP, D), jnp.float32),
        pltpu.VMEM((GROUP, D), jnp.float32),
        pltpu.VMEM((GROUP,), jnp.int32),
        pltpu.VMEM((GROUP,), jnp.int32),
        pltpu.VMEM((GROUP,), jnp.int32),
        pltpu.VMEM((GROUP,), jnp.int32),
        pltpu.VMEM((GROUP,), jnp.float32),
        pltpu.VMEM((SEG_PER_TILE,), jnp.float32),
        pltpu.VMEM_SHARED((SEG_PAD, D), jnp.float32),
        pltpu.VMEM_SHARED((SEG_PAD,), jnp.float32),
        pltpu.SemaphoreType.DMA,
        pltpu.SemaphoreType.DMA,
        pltpu.SemaphoreType.DMA,
        pltpu.SemaphoreType.DMA,
        pltpu.SemaphoreType.DMA,
        pltpu.SemaphoreType.DMA,
    ],
)(_phase1_body)


def _phase2_body(sm, cc, o):
    c = cc[...]
    cnt = jnp.transpose(c[0:1, :] + c[1:2, :], (1, 0))
    o[...] = (sm[0] + sm[1]) / jnp.maximum(cnt, 1.0)


_BS = 512

_phase2 = pl.pallas_call(
    _phase2_body,
    grid=(SEG_PAD // _BS,),
    in_specs=[
        pl.BlockSpec((NC, _BS, D), lambda i: (0, i, 0)),
        pl.BlockSpec((NC, _BS), lambda i: (0, i)),
    ],
    out_specs=pl.BlockSpec((_BS, D), lambda i: (i, 0)),
    out_shape=jax.ShapeDtypeStruct((NUM_SEG, D), jnp.float32),
)


@jax.jit
def kernel(input, segLabels):
    seg = segLabels.astype(jnp.int32)
    zsum = jnp.zeros((SEG_PAD, D), jnp.float32)
    sums, cnts = _phase1(input, seg, zsum)
    return _phase2(sums, cnts)
